# baseline (speedup 1.0000x reference)
import jax
import jax.numpy as jnp
from jax import lax
from jax.experimental import pallas as pl
from jax.experimental.pallas import tpu as pltpu
from jax.experimental.pallas import tpu_sc as plsc

_SEQ = 1729
_PATCH = 576
_B = 32
_D = 768
_LANES = 16
_NVEC = _D // _LANES
_NW = 32
_NTOK = 55
_NBUF = 4


def _sc_body(xt_hbm, pe_hbm, out_hbm, pe_v, bufs, sins, souts):
    nc = 2
    wid = lax.axis_index("s") * nc + lax.axis_index("c")
    base = wid * (_NTOK - 1)

    pltpu.sync_copy(pe_hbm, pe_v)

    def in_start(c, b):
        pltpu.make_async_copy(
            xt_hbm.at[pl.ds(base + c, 1)], bufs[b], sins[b]
        ).start()

    def in_wait(b):
        pltpu.make_async_copy(
            xt_hbm.at[pl.ds(0, 1)], bufs[b], sins[b]
        ).wait()

    def out_start_half(c, b, h):
        pltpu.make_async_copy(
            bufs[b].at[:, pl.ds(h * (_B // 2), _B // 2)],
            out_hbm.at[pl.ds(base + c, 1), pl.ds(h * (_B // 2), _B // 2)],
            souts[b],
        ).start()

    def out_wait(b):
        pltpu.make_async_copy(
            bufs[b], out_hbm.at[pl.ds(0, 1)], souts[b]
        ).wait()

    def compute_half(c, b, h, vals):
        buf = bufs[b]

        def body(j, carry):
            for k in range(_NVEC):
                plsc.addupdate(buf.at[0, j, pl.ds(k * _LANES, _LANES)], vals[k])
            return carry

        lax.fori_loop(h * (_B // 2), (h + 1) * (_B // 2), body, jnp.int32(0))

    def step(j, par, c2_valid, c2_wait):
        b2 = (par + 2) % _NBUF
        if c2_wait:
            out_wait(b2)
        if c2_valid:
            in_start(j + 2, b2)
        b = par % _NBUF
        in_wait(b)
        t = base + j
        pos = (t + _PATCH - 1) // _PATCH
        vals = [pe_v[pos, pl.ds(k * _LANES, _LANES)] for k in range(_NVEC)]
        compute_half(j, b, 0, vals)
        out_start_half(j, b, 0)
        compute_half(j, b, 1, vals)
        out_start_half(j, b, 1)

    in_start(0, 0)
    in_start(1, 1)
    step(0, 0, True, False)
    step(1, 1, True, False)

    def group(m, carry):
        j0 = 2 + 4 * m
        for u in range(4):
            step(j0 + u, 2 + u, True, True)
        return carry

    lax.fori_loop(0, 12, group, jnp.int32(0))

    for j in range(50, _NTOK):
        c2 = j + 2
        step(j, j % _NBUF, c2 < _NTOK, c2 < _NTOK)
    for j in range(_NTOK - _NBUF, _NTOK):
        out_wait(j % _NBUF)


def kernel(x, pe):
    B, S, D = x.shape
    xt = jnp.transpose(x, (1, 0, 2))
    mesh = plsc.VectorSubcoreMesh(core_axis_name="c", subcore_axis_name="s")
    sc_add = pl.kernel(
        _sc_body,
        out_type=jax.ShapeDtypeStruct((S, B, D), x.dtype),
        mesh=mesh,
        scratch_types=[
            pltpu.VMEM((4, D), jnp.float32),
            [pltpu.VMEM((1, B, D), jnp.float32) for _ in range(_NBUF)],
            [pltpu.SemaphoreType.DMA for _ in range(_NBUF)],
            [pltpu.SemaphoreType.DMA for _ in range(_NBUF)],
        ],
    )
    out_t = sc_add(xt, pe)
    return jnp.transpose(out_t, (1, 0, 2))


# speedup vs baseline: 1.0291x; 1.0291x over previous
import jax
import jax.numpy as jnp
from jax import lax
from jax.experimental import pallas as pl
from jax.experimental.pallas import tpu as pltpu
from jax.experimental.pallas import tpu_sc as plsc

_SEQ = 1729
_PATCH = 576
_B = 32
_D = 768
_LANES = 16
_NVEC = _D // _LANES
_NTOK = 54
_NBUF = 4


def _sc_body(xt_hbm, pe_hbm, out_hbm, pe_v, bufs, sins, souts):
    nc = 2
    wid = lax.axis_index("s") * nc + lax.axis_index("c")
    base = wid * _NTOK

    pltpu.sync_copy(pe_hbm, pe_v)

    def in_start(c, b):
        pltpu.make_async_copy(
            xt_hbm.at[pl.ds(base + c, 1)], bufs[b], sins[b]
        ).start()

    def in_wait(b):
        pltpu.make_async_copy(
            xt_hbm.at[pl.ds(0, 1)], bufs[b], sins[b]
        ).wait()

    def out_start(c, b):
        pltpu.make_async_copy(
            bufs[b], out_hbm.at[pl.ds(base + c, 1)], souts[b]
        ).start()

    def out_wait(b):
        pltpu.make_async_copy(
            bufs[b], out_hbm.at[pl.ds(0, 1)], souts[b]
        ).wait()

    def add_rows(b, vals, lo, hi):
        buf = bufs[b]

        def body(j, carry):
            for k in range(_NVEC):
                buf[0, j, pl.ds(k * _LANES, _LANES)] += vals[k]
            return carry

        lax.fori_loop(lo, hi, body, jnp.int32(0))

    def compute(c, b):
        t = base + c
        pos = (t + _PATCH - 1) // _PATCH
        vals = [pe_v[pos, pl.ds(k * _LANES, _LANES)] for k in range(_NVEC)]
        add_rows(b, vals, 0, _B)

    def step(j, par, c2_valid, c2_wait):
        b2 = (par + 2) % _NBUF
        if c2_wait:
            out_wait(b2)
        if c2_valid:
            in_start(j + 2, b2)
        b = par % _NBUF
        in_wait(b)
        compute(j, b)
        out_start(j, b)

    in_start(0, 0)
    in_start(1, 1)
    step(0, 0, True, False)
    step(1, 1, True, False)

    def group(m, carry):
        j0 = 2 + 4 * m
        for u in range(4):
            step(j0 + u, 2 + u, True, True)
        return carry

    lax.fori_loop(0, 12, group, jnp.int32(0))

    for j in range(50, _NTOK):
        c2 = j + 2
        step(j, j % _NBUF, c2 < _NTOK, c2 < _NTOK)
    for j in range(_NTOK - _NBUF, _NTOK):
        out_wait(j % _NBUF)

    @pl.when(wid == 31)
    def _tail():
        pltpu.sync_copy(xt_hbm.at[pl.ds(_SEQ - 1, 1)], bufs[0])
        vals = [pe_v[3, pl.ds(k * _LANES, _LANES)] for k in range(_NVEC)]
        add_rows(0, vals, 0, _B)
        pltpu.sync_copy(bufs[0], out_hbm.at[pl.ds(_SEQ - 1, 1)])


def kernel(x, pe):
    B, S, D = x.shape
    xt = jnp.transpose(x, (1, 0, 2))
    mesh = plsc.VectorSubcoreMesh(core_axis_name="c", subcore_axis_name="s")
    sc_add = pl.kernel(
        _sc_body,
        out_type=jax.ShapeDtypeStruct((S, B, D), x.dtype),
        mesh=mesh,
        scratch_types=[
            pltpu.VMEM((4, D), jnp.float32),
            [pltpu.VMEM((1, B, D), jnp.float32) for _ in range(_NBUF)],
            [pltpu.SemaphoreType.DMA for _ in range(_NBUF)],
            [pltpu.SemaphoreType.DMA for _ in range(_NBUF)],
        ],
    )
    out_t = sc_add(xt, pe)
    return jnp.transpose(out_t, (1, 0, 2))
